# baseline (speedup 1.0000x reference)
import functools

import jax
import jax.numpy as jnp
from jax import lax
from jax.experimental import pallas as pl

BATCH = 128
VOCAB = 100000
INV_TEMP = 0.2
EPS = 1e-08
ROWS_PER_BLOCK = 8


def _threefry_bits(flat_index_u32):
    x0 = jnp.zeros_like(flat_index_u32)
    x1 = flat_index_u32

    ks0 = jnp.uint32(0)
    ks1 = jnp.uint32(42)
    ks2 = jnp.uint32(0x1BD11BDA ^ 42)

    def rot(x, r):
        return (x << jnp.uint32(r)) | (x >> jnp.uint32(32 - r))

    rots = ((13, 15, 26, 6), (17, 29, 16, 24))
    keys = ((ks1, ks2), (ks2, ks0), (ks0, ks1), (ks1, ks2), (ks2, ks0))

    x0 = x0 + ks0
    x1 = x1 + ks1
    for i in range(5):
        for r in rots[i % 2]:
            x0 = x0 + x1
            x1 = rot(x1, r)
            x1 = x1 ^ x0
        x0 = x0 + keys[i][0]
        x1 = x1 + keys[i][1] + jnp.uint32(i + 1)
    return x0 ^ x1


def _gumbel_softmax_block(logits_ref, out_ref):
    rows, cols = logits_ref.shape
    pid = pl.program_id(0)
    row = lax.broadcasted_iota(jnp.uint32, (rows, cols), 0) + jnp.uint32(pid * rows)
    col = lax.broadcasted_iota(jnp.uint32, (rows, cols), 1)
    flat = row * jnp.uint32(VOCAB) + col

    bits = _threefry_bits(flat)
    u = lax.bitcast_convert_type(
        (bits >> jnp.uint32(9)) | jnp.uint32(0x3F800000), jnp.float32) - 1.0
    g = -jnp.log(-jnp.log(u + EPS) + EPS)

    z = (logits_ref[...] + g) * INV_TEMP
    m = jnp.max(z, axis=-1, keepdims=True)
    e = jnp.exp(z - m)
    s = jnp.sum(e, axis=-1, keepdims=True)
    out_ref[...] = e / s


@jax.jit
def kernel(logits):
    grid = BATCH // ROWS_PER_BLOCK
    return pl.pallas_call(
        _gumbel_softmax_block,
        grid=(grid,),
        in_specs=[pl.BlockSpec((ROWS_PER_BLOCK, VOCAB), lambda i: (i, 0))],
        out_specs=pl.BlockSpec((ROWS_PER_BLOCK, VOCAB), lambda i: (i, 0)),
        out_shape=jax.ShapeDtypeStruct((BATCH, VOCAB), jnp.float32),
    )(logits)
